# Initial kernel scaffold; baseline (speedup 1.0000x reference)
import jax
import jax.numpy as jnp
from jax.experimental import pallas as pl


def kernel(h, edge_index, edge_attr, W_fc, W_attn, W_edge, W_m):
    raise NotImplementedError("write your pallas kernel here")


# speedup vs baseline: 9.1521x; 9.1521x over previous
import functools
import jax
import jax.numpy as jnp
from jax import lax
from jax.experimental import pallas as pl
from jax.experimental.pallas import tpu as pltpu, tpu_sc as plsc

N_NODES = 10000
N_PAD = 10240
N_EDGES = 320000
NW = 32
E_W = N_EDGES // NW
NB = 125
BB = 80
NGRP = E_W // 16
ROWS_W = N_PAD // 16
D = 128


def _leaky(x):
    return jnp.where(x >= 0, x, 0.01 * x)


def _node_body(h_ref, wfc_ref, wa_ref, z_ref, s_ref):
    zb = _leaky(lax.dot_general(h_ref[...], wfc_ref[...],
                                (((1,), (1,)), ((), ())),
                                preferred_element_type=jnp.float32))
    z_ref[...] = zb
    s_ref[...] = lax.dot_general(wa_ref[...], zb,
                                 (((1,), (1,)), ((), ())),
                                 preferred_element_type=jnp.float32)


def _node_transform(h_pad, W_fc, wa12):
    R = 1024
    return pl.pallas_call(
        _node_body,
        grid=(N_PAD // R,),
        in_specs=[
            pl.BlockSpec((R, D), lambda i: (i, 0)),
            pl.BlockSpec((D, D), lambda i: (0, 0)),
            pl.BlockSpec((2, D), lambda i: (0, 0)),
        ],
        out_specs=[
            pl.BlockSpec((R, D), lambda i: (i, 0)),
            pl.BlockSpec((2, R), lambda i: (0, i)),
        ],
        out_shape=[
            jax.ShapeDtypeStruct((N_PAD, D), jnp.float32),
            jax.ShapeDtypeStruct((2, N_PAD), jnp.float32),
        ],
    )(h_pad, W_fc, wa12)


def _sc_body(z_hbm, s1_hbm, s2_hbm, srcb_hbm, dstb_hbm,
             ea_hbm, wev_hbm, wm_hbm, zrows_hbm, zden_hbm,
             outp_hbm, outd_hbm, cmax_hbm,
             sidx_v, didx_v, s1g_v, s2g_v, eab_v, buf_v,
             rows_v, mst_v, am_v, mf_v, wev_v, wm_v,
             acc_s, den_s, maxst_s, sem):
    cid = lax.axis_index("c")
    sid = lax.axis_index("s")
    wid = sid * 2 + cid

    pltpu.sync_copy(zrows_hbm, acc_s.at[pl.ds(sid * ROWS_W, ROWS_W)])
    pltpu.sync_copy(zden_hbm, den_s.at[pl.ds(sid * ROWS_W, ROWS_W)])

    pltpu.sync_copy(wev_hbm, wev_v)
    pltpu.sync_copy(wm_hbm, wm_v)

    wev = wev_v[...]
    wm = wm_v[...]

    def pass_a(b, m):
        pltpu.sync_copy(srcb_hbm.at[wid, b], sidx_v)
        pltpu.sync_copy(dstb_hbm.at[wid, b], didx_v)
        pltpu.async_copy(s1_hbm.at[sidx_v], s1g_v, sem).wait()
        pltpu.async_copy(s2_hbm.at[didx_v], s2g_v, sem).wait()
        pltpu.sync_copy(ea_hbm.at[wid, b], eab_v)
        for g in range(BB // 16):
            sl = pl.ds(g * 16, 16)
            a16 = s1g_v[sl] + s2g_v[sl]
            x = a16 * (eab_v[sl] * wev)
            ne16 = _leaky(x) * wm
            buf_v[pl.ds(b * BB + g * 16, 16)] = ne16
            m = jnp.maximum(m, ne16)
        return m

    m = lax.fori_loop(0, NB, pass_a,
                      jnp.full((16,), -jnp.inf, jnp.float32))
    mst_v[...] = m

    pltpu.sync_copy(mst_v, maxst_s.at[sid])
    plsc.subcore_barrier()
    pltpu.sync_copy(maxst_s, am_v)
    mm = am_v[0, :]
    for r in range(1, 16):
        mm = jnp.maximum(mm, am_v[r, :])
    C = jnp.max(mm)
    mf_v[...] = jnp.full((16,), C, jnp.float32)

    @pl.when(sid == 0)
    def _():
        pltpu.sync_copy(mf_v, cmax_hbm.at[cid])

    def pass_b(i, carry):
        sl = pl.ds(i * 16, 16)
        buf_v[sl] = jnp.exp(buf_v[sl] - jnp.full((16,), C, jnp.float32))
        return carry

    lax.fori_loop(0, NGRP, pass_b, 0)

    def row_batch(b, carry):
        pltpu.sync_copy(srcb_hbm.at[wid, b], sidx_v)
        pltpu.sync_copy(dstb_hbm.at[wid, b], didx_v)
        pltpu.sync_copy(buf_v.at[pl.ds(b * BB, BB)],
                        den_s.at[didx_v], add=True)
        pltpu.async_copy(z_hbm.at[sidx_v], rows_v, sem).wait()
        for r in range(BB):
            eidx = b * BB + r
            sc = plsc.load_gather(
                buf_v, [jnp.full((16,), eidx, jnp.int32)])
            for c in range(D // 16):
                cs = pl.ds(c * 16, 16)
                rows_v[r, cs] = rows_v[r, cs] * sc
        pltpu.sync_copy(rows_v, acc_s.at[didx_v], add=True)
        return carry

    lax.fori_loop(0, NB, row_batch, 0)

    plsc.subcore_barrier()

    rs = pl.ds(sid * ROWS_W, ROWS_W)
    pltpu.sync_copy(acc_s.at[rs], outp_hbm.at[cid, rs])
    pltpu.sync_copy(den_s.at[rs], outd_hbm.at[cid, rs])


def _sc_edge_softmax(z, s1, s2, srcb, dstb, ea, wev, wm, zrows, zden):
    mesh = plsc.VectorSubcoreMesh(core_axis_name="c", subcore_axis_name="s")
    f = functools.partial(
        pl.kernel, _sc_body, mesh=mesh,
        compiler_params=pltpu.CompilerParams(needs_layout_passes=False),
        out_type=[
            jax.ShapeDtypeStruct((2, N_PAD, D), jnp.float32),
            jax.ShapeDtypeStruct((2, N_PAD), jnp.float32),
            jax.ShapeDtypeStruct((2, 16), jnp.float32),
        ],
        scratch_types=[
            pltpu.VMEM((BB,), jnp.int32),
            pltpu.VMEM((BB,), jnp.int32),
            pltpu.VMEM((BB,), jnp.float32),
            pltpu.VMEM((BB,), jnp.float32),
            pltpu.VMEM((BB,), jnp.float32),
            pltpu.VMEM((E_W,), jnp.float32),
            pltpu.VMEM((BB, D), jnp.float32),
            pltpu.VMEM((16,), jnp.float32),
            pltpu.VMEM((16, 16), jnp.float32),
            pltpu.VMEM((16,), jnp.float32),
            pltpu.VMEM((16,), jnp.float32),
            pltpu.VMEM((16,), jnp.float32),
            pltpu.VMEM_SHARED((N_PAD, D), jnp.float32),
            pltpu.VMEM_SHARED((N_PAD,), jnp.float32),
            pltpu.VMEM_SHARED((16, 16), jnp.float32),
            pltpu.SemaphoreType.DMA,
        ],
    )()
    return f(z, s1, s2, srcb, dstb, ea, wev, wm, zrows, zden)


def _merge_body(p0_ref, p1_ref, dt_ref, cm_ref, o_ref):
    c0 = cm_ref[0, 0]
    c1 = cm_ref[1, 0]
    cc = jnp.maximum(c0, c1)
    f0 = jnp.exp(c0 - cc)
    f1 = jnp.exp(c1 - cc)
    num = p0_ref[...] * f0 + p1_ref[...] * f1
    den = dt_ref[:, 0:1] * f0 + dt_ref[:, 1:2] * f1
    o_ref[...] = jnp.where(den != 0, _leaky(num / den), 0.0)


def _merge(p0, p1, den_t, cm):
    R = 1024
    return pl.pallas_call(
        _merge_body,
        grid=(N_PAD // R,),
        in_specs=[
            pl.BlockSpec((R, D), lambda i: (i, 0)),
            pl.BlockSpec((R, D), lambda i: (i, 0)),
            pl.BlockSpec((R, 2), lambda i: (i, 0)),
            pl.BlockSpec((2, 16), lambda i: (0, 0)),
        ],
        out_specs=pl.BlockSpec((R, D), lambda i: (i, 0)),
        out_shape=jax.ShapeDtypeStruct((N_PAD, D), jnp.float32),
    )(p0, p1, den_t, cm)


@jax.jit
def kernel(h, edge_index, edge_attr, W_fc, W_attn, W_edge, W_m):
    src = edge_index[0].astype(jnp.int32)
    dst = edge_index[1].astype(jnp.int32)
    wa12 = W_attn.reshape(2, D)

    h_pad = jnp.pad(h, ((0, N_PAD - N_NODES), (0, 0)))
    z, s12 = _node_transform(h_pad, W_fc, wa12)
    s1 = s12[0, :N_NODES]
    s2 = s12[1, :N_NODES]

    srcb = src.reshape(NW, NB, BB)
    dstb = dst.reshape(NW, NB, BB)
    ea = edge_attr.reshape(NW, NB, BB)
    wev = jnp.full((16,), W_edge[0, 0], jnp.float32)
    wm = jnp.full((16,), W_m[0, 0], jnp.float32)
    zrows = jnp.zeros((ROWS_W, D), jnp.float32)
    zden = jnp.zeros((ROWS_W,), jnp.float32)

    outp, outd, cmax = _sc_edge_softmax(
        z, s1, s2, srcb, dstb, ea, wev, wm, zrows, zden)

    out = _merge(outp[0], outp[1], outd.T, cmax)
    return out[:N_NODES]
